# baseline (speedup 1.0000x reference)
import functools

import jax
import jax.numpy as jnp
from jax import lax
from jax.experimental import pallas as pl
from jax.experimental.pallas import tpu as pltpu
from jax.experimental.pallas import tpu_sc as plsc

N_NODES = 10000
N_EDGES = 160000
C = 256

NC = 2
NS = 16
NW = NC * NS

CP = 128
CH = 128
E_PAD = 163840
TOT = 2 * E_PAD
EPW = TOT // NW
NCHUNK = EPW // CH
RING = 4
FAST_CORE = 0
NA = 120
NB = 40

BLK_N = 1000
BLK_E = 640


def _proj_body(n_ref, w_ref, o_ref):
    p = lax.dot_general(n_ref[...], w_ref[...],
                        (((1,), (1,)), ((), ())),
                        preferred_element_type=jnp.float32)
    u = lax.bitcast_convert_type(p, jnp.uint32)
    r = u + jnp.uint32(0x7FFF) + ((u >> 16) & jnp.uint32(1))
    h = r >> 16
    packed = h[:, :CP] | (h[:, CP:] << 16)
    o_ref[...] = lax.bitcast_convert_type(packed, jnp.float32)


def _node_proj(node, W1_one):
    return pl.pallas_call(
        _proj_body,
        grid=(N_NODES // BLK_N,),
        in_specs=[
            pl.BlockSpec((BLK_N, C), lambda i: (i, 0)),
            pl.BlockSpec((C, C), lambda i: (0, 0)),
        ],
        out_specs=pl.BlockSpec((BLK_N, CP), lambda i: (i, 0)),
        out_shape=jax.ShapeDtypeStruct((N_NODES, CP), jnp.float32),
    )(node, W1_one)


def _sc_gather_body(table, idx2, out, idx_v, rows,
                    sg0, sg1, sg2, sg3, sw0, sw1, sw2, sw3):
    sg = [sg0, sg1, sg2, sg3]
    sw = [sw0, sw1, sw2, sw3]
    c = lax.axis_index("c")
    s = lax.axis_index("s")
    is_fast = c == FAST_CORE
    nch = jnp.where(is_fast, NA, NB)
    crow = jnp.where(is_fast, s * NA, NS * NA + s * NB)
    base = crow * CH

    @pl.when(is_fast)
    def _():
        pltpu.sync_copy(idx2.at[pl.ds(s * NA, NA)], idx_v)

    @pl.when(jnp.logical_not(is_fast))
    def _():
        pltpu.sync_copy(idx2.at[pl.ds(NS * NA + s * NB, NB)],
                        idx_v.at[pl.ds(0, NB)])

    def gcopy(t, b):
        return pltpu.make_async_copy(table.at[idx_v.at[t]], rows.at[b], sg[b])

    def wcopy(t, b):
        off = pl.multiple_of(base + t * CH, CH)
        return pltpu.make_async_copy(rows.at[b], out.at[pl.ds(off, CH)], sw[b])

    for b in range(RING):
        gcopy(b, b).start()

    def outer(r, carry):
        t0 = r * RING
        for b in range(RING):
            t = t0 + b
            gcopy(t, b).wait()
            wcopy(t, b).start()
            pb = (b - 1) % RING
            tn = t + RING - 1

            @pl.when(jnp.logical_and(t >= 1, tn < nch))
            def _():
                wcopy(t - 1, pb).wait()
                gcopy(tn, pb).start()
        return carry

    lax.fori_loop(0, nch // RING, outer, 0)
    for b in range(RING):
        wcopy(nch - RING + b, b).wait()


@functools.cache
def _sc_gather():
    return pl.kernel(
        _sc_gather_body,
        out_type=jax.ShapeDtypeStruct((TOT, CP), jnp.float32),
        mesh=plsc.VectorSubcoreMesh(core_axis_name="c", subcore_axis_name="s",
                                    num_cores=NC, num_subcores=NS),
        scratch_types=[
            pltpu.VMEM((NA, CH), jnp.int32),
            pltpu.VMEM((RING, CH, CP), jnp.float32),
        ] + [pltpu.SemaphoreType.DMA] * 8,
    )


def _main_body(ps_ref, pd_ref, e_ref, w2o_ref, w1t_ref, w2t_ref,
               b1o_ref, b1t_ref, b2o_ref, b2t_ref,
               g_ref, w1g_ref, b1g_ref, w2g_ref, b2g_ref, o_ref):
    dot = lambda a, b: lax.dot_general(a, b, (((1,), (1,)), ((), ())),
                                       preferred_element_type=jnp.float32)

    def unpack(ref):
        w = lax.bitcast_convert_type(ref[...], jnp.uint32)
        lo = lax.bitcast_convert_type(w << 16, jnp.float32)
        hi = lax.bitcast_convert_type(w & jnp.uint32(0xFFFF0000), jnp.float32)
        return lo, hi

    e = e_ref[...]
    psa, psb = unpack(ps_ref)
    pda, pdb = unpack(pd_ref)
    b1o = b1o_ref[...]
    h1a = jnp.maximum(psa + pda + b1o[:, :CP], 0.0)
    h1b = jnp.maximum(psb + pdb + b1o[:, CP:], 0.0)
    w2o = w2o_ref[...]
    t = dot(h1a, w2o[:, :CP]) + dot(h1b, w2o[:, CP:])
    h2 = jnp.maximum(dot(e, w1t_ref[...]) + b1t_ref[...], 0.0)
    t = t + dot(h2, w2t_ref[...])
    hg = dot(jnp.maximum(dot(g_ref[...], w1g_ref[...]) + b1g_ref[...], 0.0),
             w2g_ref[...]) + b2g_ref[...]
    s = t + b2o_ref[...] + b2t_ref[...] + hg
    m = jnp.mean(s, axis=1, keepdims=True)
    v = jnp.mean((s - m) ** 2, axis=1, keepdims=True)
    sn = (s - m) * lax.rsqrt(v + 1e-5)
    o_ref[...] = e + jnp.maximum(sn, 0.0)


def _main(pboth, edge, W2_one, W1_two, W2_two, b1o, b1t, b2o, b2t,
          g, W1_three, b1g, W2_three, b2g):
    full = lambda: pl.BlockSpec((C, C), lambda i: (0, 0))
    vec = lambda: pl.BlockSpec((1, C), lambda i: (0, 0))
    doff = E_PAD // BLK_E
    return pl.pallas_call(
        _main_body,
        grid=(N_EDGES // BLK_E,),
        in_specs=[
            pl.BlockSpec((BLK_E, CP), lambda i: (i, 0)),
            pl.BlockSpec((BLK_E, CP), lambda i: (i + doff, 0)),
            pl.BlockSpec((BLK_E, C), lambda i: (i, 0)),
            full(), full(), full(),
            vec(), vec(), vec(), vec(),
            vec(), full(), vec(), full(), vec(),
        ],
        out_specs=pl.BlockSpec((BLK_E, C), lambda i: (i, 0)),
        out_shape=jax.ShapeDtypeStruct((N_EDGES, C), jnp.float32),
    )(pboth, pboth, edge, W2_one, W1_two, W2_two, b1o, b1t, b2o, b2t,
      g, W1_three, b1g, W2_three, b2g)


def kernel(node_features, edge_index, edge_features, global_features,
           W1_one, b1_one, W2_one, b2_one,
           W1_two, b1_two, W2_two, b2_two,
           W1_three, b1_three, W2_three, b2_three):
    node = node_features[0]
    edge = edge_features[0]
    src = edge_index[0, 0]
    dst = edge_index[0, 1]
    zpad = jnp.zeros((E_PAD - N_EDGES,), jnp.int32)
    idx_all = jnp.concatenate([src, zpad, dst, zpad])

    P = _node_proj(node, W1_one)
    pboth = _sc_gather()(P, idx_all.reshape(TOT // CH, CH))

    r2 = lambda b: b.reshape(1, C)
    out = _main(pboth, edge, W2_one, W1_two, W2_two,
                r2(b1_one), r2(b1_two), r2(b2_one), r2(b2_two),
                global_features[0], W1_three, r2(b1_three), W2_three,
                r2(b2_three))
    return lax.stop_gradient(out[None])


# speedup vs baseline: 1.0011x; 1.0011x over previous
import functools

import jax
import jax.numpy as jnp
from jax import lax
from jax.experimental import pallas as pl
from jax.experimental.pallas import tpu as pltpu
from jax.experimental.pallas import tpu_sc as plsc

N_NODES = 10000
N_EDGES = 160000
C = 256

NC = 2
NS = 16
NW = NC * NS

CP = 128
CH = 128
E_PAD = 163840
TOT = 2 * E_PAD
EPW = TOT // NW
NCHUNK = EPW // CH
RING = 4
FAST_CORE = 1
NA = 120
NB = 40

BLK_N = 1000
BLK_E = 640


def _proj_body(n_ref, w_ref, o_ref):
    p = lax.dot_general(n_ref[...], w_ref[...],
                        (((1,), (1,)), ((), ())),
                        preferred_element_type=jnp.float32)
    u = lax.bitcast_convert_type(p, jnp.uint32)
    r = u + jnp.uint32(0x7FFF) + ((u >> 16) & jnp.uint32(1))
    h = r >> 16
    packed = h[:, :CP] | (h[:, CP:] << 16)
    o_ref[...] = lax.bitcast_convert_type(packed, jnp.float32)


def _node_proj(node, W1_one):
    return pl.pallas_call(
        _proj_body,
        grid=(N_NODES // BLK_N,),
        in_specs=[
            pl.BlockSpec((BLK_N, C), lambda i: (i, 0)),
            pl.BlockSpec((C, C), lambda i: (0, 0)),
        ],
        out_specs=pl.BlockSpec((BLK_N, CP), lambda i: (i, 0)),
        out_shape=jax.ShapeDtypeStruct((N_NODES, CP), jnp.float32),
    )(node, W1_one)


def _sc_gather_body(table, idx2, out, idx_v, rows,
                    sg0, sg1, sg2, sg3, sw0, sw1, sw2, sw3):
    sg = [sg0, sg1, sg2, sg3]
    sw = [sw0, sw1, sw2, sw3]
    c = lax.axis_index("c")
    s = lax.axis_index("s")
    is_fast = c == FAST_CORE
    nch = jnp.where(is_fast, NA, NB)
    crow = jnp.where(is_fast, s * NA, NS * NA + s * NB)
    base = crow * CH

    @pl.when(is_fast)
    def _():
        pltpu.sync_copy(idx2.at[pl.ds(s * NA, NA)], idx_v)

    @pl.when(jnp.logical_not(is_fast))
    def _():
        pltpu.sync_copy(idx2.at[pl.ds(NS * NA + s * NB, NB)],
                        idx_v.at[pl.ds(0, NB)])

    def gcopy(t, b):
        return pltpu.make_async_copy(table.at[idx_v.at[t]], rows.at[b], sg[b])

    def wcopy(t, b):
        off = pl.multiple_of(base + t * CH, CH)
        return pltpu.make_async_copy(rows.at[b], out.at[pl.ds(off, CH)], sw[b])

    for b in range(RING):
        gcopy(b, b).start()

    def outer(r, carry):
        t0 = r * RING
        for b in range(RING):
            t = t0 + b
            gcopy(t, b).wait()
            wcopy(t, b).start()
            pb = (b - 1) % RING
            tn = t + RING - 1

            @pl.when(jnp.logical_and(t >= 1, tn < nch))
            def _():
                wcopy(t - 1, pb).wait()
                gcopy(tn, pb).start()
        return carry

    lax.fori_loop(0, nch // RING, outer, 0)
    for b in range(RING):
        wcopy(nch - RING + b, b).wait()


@functools.cache
def _sc_gather():
    return pl.kernel(
        _sc_gather_body,
        out_type=jax.ShapeDtypeStruct((TOT, CP), jnp.float32),
        mesh=plsc.VectorSubcoreMesh(core_axis_name="c", subcore_axis_name="s",
                                    num_cores=NC, num_subcores=NS),
        scratch_types=[
            pltpu.VMEM((NA, CH), jnp.int32),
            pltpu.VMEM((RING, CH, CP), jnp.float32),
        ] + [pltpu.SemaphoreType.DMA] * 8,
    )


def _main_body(ps_ref, pd_ref, e_ref, w2o_ref, w1t_ref, w2t_ref,
               b1o_ref, b1t_ref, b2o_ref, b2t_ref,
               g_ref, w1g_ref, b1g_ref, w2g_ref, b2g_ref, o_ref):
    dot = lambda a, b: lax.dot_general(a, b, (((1,), (1,)), ((), ())),
                                       preferred_element_type=jnp.float32)

    def unpack(ref):
        w = lax.bitcast_convert_type(ref[...], jnp.uint32)
        lo = lax.bitcast_convert_type(w << 16, jnp.float32)
        hi = lax.bitcast_convert_type(w & jnp.uint32(0xFFFF0000), jnp.float32)
        return lo, hi

    e = e_ref[...]
    psa, psb = unpack(ps_ref)
    pda, pdb = unpack(pd_ref)
    b1o = b1o_ref[...]
    h1a = jnp.maximum(psa + pda + b1o[:, :CP], 0.0)
    h1b = jnp.maximum(psb + pdb + b1o[:, CP:], 0.0)
    w2o = w2o_ref[...]
    t = dot(h1a, w2o[:, :CP]) + dot(h1b, w2o[:, CP:])
    h2 = jnp.maximum(dot(e, w1t_ref[...]) + b1t_ref[...], 0.0)
    t = t + dot(h2, w2t_ref[...])
    hg = dot(jnp.maximum(dot(g_ref[...], w1g_ref[...]) + b1g_ref[...], 0.0),
             w2g_ref[...]) + b2g_ref[...]
    s = t + b2o_ref[...] + b2t_ref[...] + hg
    m = jnp.mean(s, axis=1, keepdims=True)
    v = jnp.mean((s - m) ** 2, axis=1, keepdims=True)
    sn = (s - m) * lax.rsqrt(v + 1e-5)
    o_ref[...] = e + jnp.maximum(sn, 0.0)


def _main(pboth, edge, W2_one, W1_two, W2_two, b1o, b1t, b2o, b2t,
          g, W1_three, b1g, W2_three, b2g):
    full = lambda: pl.BlockSpec((C, C), lambda i: (0, 0))
    vec = lambda: pl.BlockSpec((1, C), lambda i: (0, 0))
    doff = E_PAD // BLK_E
    return pl.pallas_call(
        _main_body,
        grid=(N_EDGES // BLK_E,),
        in_specs=[
            pl.BlockSpec((BLK_E, CP), lambda i: (i, 0)),
            pl.BlockSpec((BLK_E, CP), lambda i: (i + doff, 0)),
            pl.BlockSpec((BLK_E, C), lambda i: (i, 0)),
            full(), full(), full(),
            vec(), vec(), vec(), vec(),
            vec(), full(), vec(), full(), vec(),
        ],
        out_specs=pl.BlockSpec((BLK_E, C), lambda i: (i, 0)),
        out_shape=jax.ShapeDtypeStruct((N_EDGES, C), jnp.float32),
    )(pboth, pboth, edge, W2_one, W1_two, W2_two, b1o, b1t, b2o, b2t,
      g, W1_three, b1g, W2_three, b2g)


def kernel(node_features, edge_index, edge_features, global_features,
           W1_one, b1_one, W2_one, b2_one,
           W1_two, b1_two, W2_two, b2_two,
           W1_three, b1_three, W2_three, b2_three):
    node = node_features[0]
    edge = edge_features[0]
    src = edge_index[0, 0]
    dst = edge_index[0, 1]
    zpad = jnp.zeros((E_PAD - N_EDGES,), jnp.int32)
    idx_all = jnp.concatenate([src, zpad, dst, zpad])

    P = _node_proj(node, W1_one)
    pboth = _sc_gather()(P, idx_all.reshape(TOT // CH, CH))

    r2 = lambda b: b.reshape(1, C)
    out = _main(pboth, edge, W2_one, W1_two, W2_two,
                r2(b1_one), r2(b1_two), r2(b2_one), r2(b2_two),
                global_features[0], W1_three, r2(b1_three), W2_three,
                r2(b2_three))
    return lax.stop_gradient(out[None])


# speedup vs baseline: 1.0636x; 1.0624x over previous
import functools

import jax
import jax.numpy as jnp
from jax import lax
from jax.experimental import pallas as pl
from jax.experimental.pallas import tpu as pltpu
from jax.experimental.pallas import tpu_sc as plsc

N_NODES = 10000
N_EDGES = 160000
C = 256

NC = 2
NS = 16
NW = NC * NS

CP = 128
CH = 128
E_PAD = 163840
TOT = 2 * E_PAD
EPW = TOT // NW
NCHUNK = EPW // CH
RING = 4
FAST_CORE = 1
NA = NCHUNK
NB = NCHUNK

BLK_N = 1000
BLK_E = 640


def _proj_body(n_ref, w_ref, o_ref):
    p = lax.dot_general(n_ref[...], w_ref[...],
                        (((1,), (1,)), ((), ())),
                        preferred_element_type=jnp.float32)
    u = lax.bitcast_convert_type(p, jnp.uint32)
    r = u + jnp.uint32(0x7FFF) + ((u >> 16) & jnp.uint32(1))
    h = r >> 16
    packed = h[:, :CP] | (h[:, CP:] << 16)
    o_ref[...] = lax.bitcast_convert_type(packed, jnp.float32)


def _node_proj(node, W1_one):
    return pl.pallas_call(
        _proj_body,
        grid=(N_NODES // BLK_N,),
        in_specs=[
            pl.BlockSpec((BLK_N, C), lambda i: (i, 0)),
            pl.BlockSpec((C, C), lambda i: (0, 0)),
        ],
        out_specs=pl.BlockSpec((BLK_N, CP), lambda i: (i, 0)),
        out_shape=jax.ShapeDtypeStruct((N_NODES, CP), jnp.float32),
    )(node, W1_one)


def _sc_gather_body(table, idx2, out, idx_v, rows,
                    sg0, sg1, sg2, sg3, sw0, sw1, sw2, sw3):
    sg = [sg0, sg1, sg2, sg3]
    sw = [sw0, sw1, sw2, sw3]
    c = lax.axis_index("c")
    s = lax.axis_index("s")
    is_fast = c == FAST_CORE
    nch = jnp.where(is_fast, NA, NB)
    crow = jnp.where(is_fast, s * NA, NS * NA + s * NB)
    base = crow * CH

    @pl.when(is_fast)
    def _():
        pltpu.sync_copy(idx2.at[pl.ds(s * NA, NA)], idx_v)

    @pl.when(jnp.logical_not(is_fast))
    def _():
        pltpu.sync_copy(idx2.at[pl.ds(NS * NA + s * NB, NB)],
                        idx_v.at[pl.ds(0, NB)])

    def gcopy(t, b):
        return pltpu.make_async_copy(table.at[idx_v.at[t]], rows.at[b], sg[b])

    def wcopy(t, b):
        off = pl.multiple_of(base + t * CH, CH)
        return pltpu.make_async_copy(rows.at[b], out.at[pl.ds(off, CH)], sw[b])

    for b in range(RING):
        gcopy(b, b).start()

    def outer(r, carry):
        t0 = r * RING
        for b in range(RING):
            t = t0 + b
            gcopy(t, b).wait()
            wcopy(t, b).start()
            pb = (b - 1) % RING
            tn = t + RING - 1

            @pl.when(jnp.logical_and(t >= 1, tn < nch))
            def _():
                wcopy(t - 1, pb).wait()
                gcopy(tn, pb).start()
        return carry

    lax.fori_loop(0, nch // RING, outer, 0)
    for b in range(RING):
        wcopy(nch - RING + b, b).wait()


@functools.cache
def _sc_gather():
    return pl.kernel(
        _sc_gather_body,
        out_type=jax.ShapeDtypeStruct((TOT, CP), jnp.float32),
        mesh=plsc.VectorSubcoreMesh(core_axis_name="c", subcore_axis_name="s",
                                    num_cores=NC, num_subcores=NS),
        scratch_types=[
            pltpu.VMEM((NA, CH), jnp.int32),
            pltpu.VMEM((RING, CH, CP), jnp.float32),
        ] + [pltpu.SemaphoreType.DMA] * 8,
    )


def _main_body(ps_ref, pd_ref, e_ref, w2o_ref, w1t_ref, w2t_ref,
               b1o_ref, b1t_ref, b2o_ref, b2t_ref,
               g_ref, w1g_ref, b1g_ref, w2g_ref, b2g_ref, o_ref):
    dot = lambda a, b: lax.dot_general(a, b, (((1,), (1,)), ((), ())),
                                       preferred_element_type=jnp.float32)

    def unpack(ref):
        w = lax.bitcast_convert_type(ref[...], jnp.uint32)
        lo = lax.bitcast_convert_type(w << 16, jnp.float32)
        hi = lax.bitcast_convert_type(w & jnp.uint32(0xFFFF0000), jnp.float32)
        return lo, hi

    bf = jnp.bfloat16
    e = e_ref[...]
    psa, psb = unpack(ps_ref)
    pda, pdb = unpack(pd_ref)
    b1o = b1o_ref[...]
    h1a = jnp.maximum(psa + pda + b1o[:, :CP], 0.0)
    h1b = jnp.maximum(psb + pdb + b1o[:, CP:], 0.0)
    w2o = w2o_ref[...]
    t = dot(h1a.astype(bf), w2o[:, :CP].astype(bf))
    t = t + dot(h1b.astype(bf), w2o[:, CP:].astype(bf))
    h2 = jnp.maximum(dot(e.astype(bf), w1t_ref[...].astype(bf))
                     + b1t_ref[...], 0.0)
    t = t + dot(h2.astype(bf), w2t_ref[...].astype(bf))
    hg = dot(jnp.maximum(dot(g_ref[...], w1g_ref[...]) + b1g_ref[...], 0.0),
             w2g_ref[...]) + b2g_ref[...]
    s = t + b2o_ref[...] + b2t_ref[...] + hg
    m = jnp.mean(s, axis=1, keepdims=True)
    v = jnp.mean((s - m) ** 2, axis=1, keepdims=True)
    sn = (s - m) * lax.rsqrt(v + 1e-5)
    o_ref[...] = e + jnp.maximum(sn, 0.0)


def _main(pboth, edge, W2_one, W1_two, W2_two, b1o, b1t, b2o, b2t,
          g, W1_three, b1g, W2_three, b2g):
    full = lambda: pl.BlockSpec((C, C), lambda i: (0, 0))
    vec = lambda: pl.BlockSpec((1, C), lambda i: (0, 0))
    doff = E_PAD // BLK_E
    return pl.pallas_call(
        _main_body,
        grid=(N_EDGES // BLK_E,),
        in_specs=[
            pl.BlockSpec((BLK_E, CP), lambda i: (i, 0)),
            pl.BlockSpec((BLK_E, CP), lambda i: (i + doff, 0)),
            pl.BlockSpec((BLK_E, C), lambda i: (i, 0)),
            full(), full(), full(),
            vec(), vec(), vec(), vec(),
            vec(), full(), vec(), full(), vec(),
        ],
        out_specs=pl.BlockSpec((BLK_E, C), lambda i: (i, 0)),
        out_shape=jax.ShapeDtypeStruct((N_EDGES, C), jnp.float32),
    )(pboth, pboth, edge, W2_one, W1_two, W2_two, b1o, b1t, b2o, b2t,
      g, W1_three, b1g, W2_three, b2g)


def kernel(node_features, edge_index, edge_features, global_features,
           W1_one, b1_one, W2_one, b2_one,
           W1_two, b1_two, W2_two, b2_two,
           W1_three, b1_three, W2_three, b2_three):
    node = node_features[0]
    edge = edge_features[0]
    src = edge_index[0, 0]
    dst = edge_index[0, 1]
    zpad = jnp.zeros((E_PAD - N_EDGES,), jnp.int32)
    idx_all = jnp.concatenate([src, zpad, dst, zpad])

    P = _node_proj(node, W1_one)
    pboth = _sc_gather()(P, idx_all.reshape(TOT // CH, CH))

    r2 = lambda b: b.reshape(1, C)
    out = _main(pboth, edge, W2_one, W1_two, W2_two,
                r2(b1_one), r2(b1_two), r2(b2_one), r2(b2_two),
                global_features[0], W1_three, r2(b1_three), W2_three,
                r2(b2_three))
    return lax.stop_gradient(out[None])


# speedup vs baseline: 1.1971x; 1.1256x over previous
import functools

import jax
import jax.numpy as jnp
from jax import lax
from jax.experimental import pallas as pl
from jax.experimental.pallas import tpu as pltpu
from jax.experimental.pallas import tpu_sc as plsc

N_NODES = 10000
N_EDGES = 160000
C = 256

NC = 2
NS = 16
NW = NC * NS

CP = 128
CH = 128
E_PAD = 163840
TOT = 2 * E_PAD
EPW = TOT // NW
NCHUNK = EPW // CH
RING = 4
FAST_CORE = 1
NA = NCHUNK
NB = NCHUNK

BLK_N = 1000
BLK_E = 1280


def _proj_body(n_ref, w_ref, o_ref):
    p = lax.dot_general(n_ref[...], w_ref[...],
                        (((1,), (1,)), ((), ())),
                        preferred_element_type=jnp.float32)
    u = lax.bitcast_convert_type(p, jnp.uint32)
    r = u + jnp.uint32(0x7FFF) + ((u >> 16) & jnp.uint32(1))
    h = r >> 16
    packed = h[:, :CP] | (h[:, CP:] << 16)
    o_ref[...] = lax.bitcast_convert_type(packed, jnp.float32)


def _node_proj(node, W1_one):
    return pl.pallas_call(
        _proj_body,
        grid=(N_NODES // BLK_N,),
        in_specs=[
            pl.BlockSpec((BLK_N, C), lambda i: (i, 0)),
            pl.BlockSpec((C, C), lambda i: (0, 0)),
        ],
        out_specs=pl.BlockSpec((BLK_N, CP), lambda i: (i, 0)),
        out_shape=jax.ShapeDtypeStruct((N_NODES, CP), jnp.float32),
    )(node, W1_one)


def _sc_gather_body(table, idx2, out, idx_v, rows,
                    sg0, sg1, sg2, sg3, sw0, sw1, sw2, sw3):
    sg = [sg0, sg1, sg2, sg3]
    sw = [sw0, sw1, sw2, sw3]
    c = lax.axis_index("c")
    s = lax.axis_index("s")
    is_fast = c == FAST_CORE
    nch = jnp.where(is_fast, NA, NB)
    crow = jnp.where(is_fast, s * NA, NS * NA + s * NB)
    base = crow * CH

    @pl.when(is_fast)
    def _():
        pltpu.sync_copy(idx2.at[pl.ds(s * NA, NA)], idx_v)

    @pl.when(jnp.logical_not(is_fast))
    def _():
        pltpu.sync_copy(idx2.at[pl.ds(NS * NA + s * NB, NB)],
                        idx_v.at[pl.ds(0, NB)])

    def gcopy(t, b):
        return pltpu.make_async_copy(table.at[idx_v.at[t]], rows.at[b], sg[b])

    def wcopy(t, b):
        off = pl.multiple_of(base + t * CH, CH)
        return pltpu.make_async_copy(rows.at[b], out.at[pl.ds(off, CH)], sw[b])

    for b in range(RING):
        gcopy(b, b).start()

    def outer(r, carry):
        t0 = r * RING
        for b in range(RING):
            t = t0 + b
            gcopy(t, b).wait()
            wcopy(t, b).start()
            pb = (b - 1) % RING
            tn = t + RING - 1

            @pl.when(jnp.logical_and(t >= 1, tn < nch))
            def _():
                wcopy(t - 1, pb).wait()
                gcopy(tn, pb).start()
        return carry

    lax.fori_loop(0, nch // RING, outer, 0)
    for b in range(RING):
        wcopy(nch - RING + b, b).wait()


@functools.cache
def _sc_gather():
    return pl.kernel(
        _sc_gather_body,
        out_type=jax.ShapeDtypeStruct((TOT, CP), jnp.float32),
        mesh=plsc.VectorSubcoreMesh(core_axis_name="c", subcore_axis_name="s",
                                    num_cores=NC, num_subcores=NS),
        scratch_types=[
            pltpu.VMEM((NA, CH), jnp.int32),
            pltpu.VMEM((RING, CH, CP), jnp.float32),
        ] + [pltpu.SemaphoreType.DMA] * 8,
    )


def _main_body(ps_ref, pd_ref, e_ref, w2o_ref, w1t_ref, w2t_ref,
               b1o_ref, b1t_ref, b2o_ref, b2t_ref,
               g_ref, w1g_ref, b1g_ref, w2g_ref, b2g_ref, o_ref):
    dot = lambda a, b: lax.dot_general(a, b, (((1,), (1,)), ((), ())),
                                       preferred_element_type=jnp.float32)

    def unpack(ref):
        w = lax.bitcast_convert_type(ref[...], jnp.uint32)
        lo = lax.bitcast_convert_type(w << 16, jnp.float32)
        hi = lax.bitcast_convert_type(w & jnp.uint32(0xFFFF0000), jnp.float32)
        return lo, hi

    bf = jnp.bfloat16
    e = e_ref[...]
    psa, psb = unpack(ps_ref)
    pda, pdb = unpack(pd_ref)
    b1o = b1o_ref[...]
    h1a = jnp.maximum(psa + pda + b1o[:, :CP], 0.0)
    h1b = jnp.maximum(psb + pdb + b1o[:, CP:], 0.0)
    w2o = w2o_ref[...]
    t = dot(h1a.astype(bf), w2o[:, :CP].astype(bf))
    t = t + dot(h1b.astype(bf), w2o[:, CP:].astype(bf))
    h2 = jnp.maximum(dot(e.astype(bf), w1t_ref[...].astype(bf))
                     + b1t_ref[...], 0.0)
    t = t + dot(h2.astype(bf), w2t_ref[...].astype(bf))
    hg = dot(jnp.maximum(dot(g_ref[...], w1g_ref[...]) + b1g_ref[...], 0.0),
             w2g_ref[...]) + b2g_ref[...]
    s = t + b2o_ref[...] + b2t_ref[...] + hg
    m = jnp.mean(s, axis=1, keepdims=True)
    v = jnp.mean((s - m) ** 2, axis=1, keepdims=True)
    sn = (s - m) * lax.rsqrt(v + 1e-5)
    o_ref[...] = e + jnp.maximum(sn, 0.0)


def _main(pboth, edge, W2_one, W1_two, W2_two, b1o, b1t, b2o, b2t,
          g, W1_three, b1g, W2_three, b2g):
    full = lambda: pl.BlockSpec((C, C), lambda i: (0, 0))
    vec = lambda: pl.BlockSpec((1, C), lambda i: (0, 0))
    doff = E_PAD // BLK_E
    return pl.pallas_call(
        _main_body,
        grid=(N_EDGES // BLK_E,),
        in_specs=[
            pl.BlockSpec((BLK_E, CP), lambda i: (i, 0)),
            pl.BlockSpec((BLK_E, CP), lambda i: (i + doff, 0)),
            pl.BlockSpec((BLK_E, C), lambda i: (i, 0)),
            full(), full(), full(),
            vec(), vec(), vec(), vec(),
            vec(), full(), vec(), full(), vec(),
        ],
        out_specs=pl.BlockSpec((BLK_E, C), lambda i: (i, 0)),
        out_shape=jax.ShapeDtypeStruct((N_EDGES, C), jnp.float32),
    )(pboth, pboth, edge, W2_one, W1_two, W2_two, b1o, b1t, b2o, b2t,
      g, W1_three, b1g, W2_three, b2g)


def kernel(node_features, edge_index, edge_features, global_features,
           W1_one, b1_one, W2_one, b2_one,
           W1_two, b1_two, W2_two, b2_two,
           W1_three, b1_three, W2_three, b2_three):
    node = node_features[0]
    edge = edge_features[0]
    src = edge_index[0, 0]
    dst = edge_index[0, 1]
    zpad = jnp.zeros((E_PAD - N_EDGES,), jnp.int32)
    idx_all = jnp.concatenate([src, zpad, dst, zpad])

    P = _node_proj(node, W1_one)
    pboth = _sc_gather()(P, idx_all.reshape(TOT // CH, CH))

    r2 = lambda b: b.reshape(1, C)
    out = _main(pboth, edge, W2_one, W1_two, W2_two,
                r2(b1_one), r2(b1_two), r2(b2_one), r2(b2_two),
                global_features[0], W1_three, r2(b1_three), W2_three,
                r2(b2_three))
    return lax.stop_gradient(out[None])
